# baseline (speedup 1.0000x reference)
import functools

import jax
import jax.numpy as jnp
from jax import lax
from jax.experimental import pallas as pl
from jax.experimental.pallas import tpu as pltpu
from jax.experimental.pallas import tpu_sc as plsc

_B, _C, _H, _W = 8, 19, 512, 512
_HB = 32
_H_TC = 480
assert _H_TC % _HB == 0

_NEG_LOG2E = -1.4426950408889634
_LN2 = 0.6931471805599453

_LP = (0.03044900453868939, -0.13158182508881333, 0.28527268109062165,
       -0.4902307234234269, 0.9992354838332771, 9.975032552123064e-06)

_LP4 = (-0.055459313742082655, 0.21866548366222538, -0.46644243862756857,
        0.9962619482337944, 6.94457445418184e-05)


def _tc_body(x_ref, t_ref, out_ref):
    def chunk(i, carry):
        acc1, acc2 = carry
        r = i * 8
        t = t_ref[0, pl.ds(r, 8), :]
        for c in range(_C):
            xc = x_ref[0, c, pl.ds(r, 8), :]
            a = jnp.abs(xc)
            zpa = jnp.where(t == c, a - xc, a + xc)
            u = jnp.exp2(a * _NEG_LOG2E)
            l = jnp.log2(1.0 + u)
            acc1 = acc1 + zpa
            acc2 = acc2 + l
        return acc1, acc2

    z = jnp.zeros((8, _W), jnp.float32)
    acc1, acc2 = jax.lax.fori_loop(0, _HB // 8, chunk, (z, z))
    s = 0.5 * jnp.sum(acc1) + _LN2 * jnp.sum(acc2)

    @pl.when((pl.program_id(0) == 0) & (pl.program_id(1) == 0))
    def _():
        out_ref[0, 0] = 0.0

    out_ref[0, 0] += s


_info = plsc.get_sparse_core_info()
_NC, _NS, _L = _info.num_cores, _info.num_subcores, _info.num_lanes
_NW = _NC * _NS
_HSC = _H - _H_TC
_NPAIRS = _B * _HSC
_PPW = _NPAIRS // _NW

_sc_mesh = plsc.VectorSubcoreMesh(core_axis_name="c", subcore_axis_name="s")


@functools.partial(
    pl.kernel,
    mesh=_sc_mesh,
    compiler_params=pltpu.CompilerParams(needs_layout_passes=False),
    out_type=jax.ShapeDtypeStruct((_NW, _L), jnp.float32),
    scratch_types=[
        pltpu.VMEM((_C, _W), jnp.float32),
        pltpu.VMEM((_C, _W), jnp.float32),
        pltpu.VMEM((_W,), jnp.int32),
        pltpu.VMEM((_W,), jnp.int32),
        pltpu.VMEM((_L,), jnp.float32),
        pltpu.SemaphoreType.DMA((2, 2)),
    ],
)
def _sc_kernel(x_hbm, t_hbm, out_hbm, x_v0, x_v1, t_v0, t_v1, res_v, sems):
    wid = lax.axis_index("s") * _NC + lax.axis_index("c")
    x_bufs, t_bufs = (x_v0, x_v1), (t_v0, t_v1)

    def issue(j, slot):
        p = j * _NW + wid
        b = lax.bitwise_and(p, _B - 1)
        h = lax.shift_right_logical(p, 3) + _H_TC
        cx = pltpu.async_copy(x_hbm.at[b, :, h, :], x_bufs[slot], sems.at[slot, 0])
        ct = pltpu.async_copy(t_hbm.at[b, h, :], t_bufs[slot], sems.at[slot, 1])
        return cx, ct

    lanes = lax.broadcasted_iota(jnp.int32, (_L,), 0)

    def compute(slot, carry):
        x_v, t_v = x_bufs[slot], t_bufs[slot]

        def chunk(i, carry2):
            a1s, a2s, a3 = carry2
            a1s, a2s = list(a1s), list(a2s)
            col = lanes + i * _L
            tw = t_v[pl.ds(i * _L, _L)]
            xg = plsc.load_gather(x_v, [tw, col])
            a3 = a3 + xg
            for c in range(_C):
                xv = x_v[c, pl.ds(i * _L, _L)]
                nx = -xv
                na = jnp.minimum(xv, nx)
                u = jnp.exp(na)
                pp = _LP4[0]
                for k in _LP4[1:]:
                    pp = pp * u + k
                a1s[c & 3] = a1s[c & 3] + (xv - na)
                a2s[c & 3] = a2s[c & 3] + pp
            return tuple(a1s), tuple(a2s), a3

        return lax.fori_loop(0, _W // _L, chunk, carry)

    z = jnp.zeros((_L,), jnp.float32)
    carry = ((z, z, z, z), (z, z, z, z), z)
    pending = issue(0, 0)
    for j in range(_PPW):
        nxt = issue(j + 1, (j + 1) & 1) if j + 1 < _PPW else None
        pending[0].wait()
        pending[1].wait()
        carry = compute(j & 1, carry)
        pending = nxt
    a1s, a2s, a3 = carry
    acc1 = (a1s[0] + a1s[1]) + (a1s[2] + a1s[3])
    acc2 = (a2s[0] + a2s[1]) + (a2s[2] + a2s[3])
    res_v[...] = acc1 * 0.5 - a3 + acc2
    pltpu.sync_copy(res_v, out_hbm.at[wid])


def kernel(input, target, epoch):
    del epoch
    n = input.size
    tc_out = pl.pallas_call(
        _tc_body,
        grid=(_B, _H_TC // _HB),
        in_specs=[
            pl.BlockSpec((1, _C, _HB, _W), lambda b, h: (b, 0, h, 0)),
            pl.BlockSpec((1, _HB, _W), lambda b, h: (b, h, 0)),
        ],
        out_specs=pl.BlockSpec(memory_space=pltpu.SMEM),
        out_shape=jax.ShapeDtypeStruct((1, 1), jnp.float32),
    )(input, target)
    sc_out = _sc_kernel(input, target)
    return (tc_out[0, 0] + jnp.sum(sc_out)) / n


# speedup vs baseline: 1.0363x; 1.0363x over previous
import functools

import jax
import jax.numpy as jnp
from jax import lax
from jax.experimental import pallas as pl
from jax.experimental.pallas import tpu as pltpu
from jax.experimental.pallas import tpu_sc as plsc

_B, _C, _H, _W = 8, 19, 512, 512
_HB = 32
_H_TC = 480
assert _H_TC % _HB == 0

_NEG_LOG2E = -1.4426950408889634
_LN2 = 0.6931471805599453

_LP = (0.03044900453868939, -0.13158182508881333, 0.28527268109062165,
       -0.4902307234234269, 0.9992354838332771, 9.975032552123064e-06)

_LP4 = (-0.055459313742082655, 0.21866548366222538, -0.46644243862756857,
        0.9962619482337944, 6.94457445418184e-05)


def _tc_body(x_ref, t_ref, out_ref):
    def chunk(i, carry):
        acc1, acc2 = carry
        r = i * 8
        t = t_ref[0, pl.ds(r, 8), :]
        for c in range(_C):
            xc = x_ref[0, c, pl.ds(r, 8), :]
            a = jnp.abs(xc)
            zpa = jnp.where(t == c, a - xc, a + xc)
            u = jnp.exp2(a * _NEG_LOG2E)
            l = jnp.log2(1.0 + u)
            acc1 = acc1 + zpa
            acc2 = acc2 + l
        return acc1, acc2

    z = jnp.zeros((8, _W), jnp.float32)
    acc1, acc2 = jax.lax.fori_loop(0, _HB // 8, chunk, (z, z))

    @pl.when((pl.program_id(0) == 0) & (pl.program_id(1) == 0))
    def _():
        out_ref[...] = jnp.zeros_like(out_ref)

    out_ref[...] += 0.5 * acc1 + _LN2 * acc2


_info = plsc.get_sparse_core_info()
_NC, _NS, _L = _info.num_cores, _info.num_subcores, _info.num_lanes
_NW = _NC * _NS
_HSC = _H - _H_TC
_NPAIRS = _B * _HSC
_PPW = _NPAIRS // _NW

_sc_mesh = plsc.VectorSubcoreMesh(core_axis_name="c", subcore_axis_name="s")


@functools.partial(
    pl.kernel,
    mesh=_sc_mesh,
    compiler_params=pltpu.CompilerParams(needs_layout_passes=False),
    out_type=jax.ShapeDtypeStruct((_NW, _L), jnp.float32),
    scratch_types=[
        pltpu.VMEM((_C, _W), jnp.float32),
        pltpu.VMEM((_C, _W), jnp.float32),
        pltpu.VMEM((_W,), jnp.int32),
        pltpu.VMEM((_W,), jnp.int32),
        pltpu.VMEM((_L,), jnp.float32),
        pltpu.SemaphoreType.DMA((2, 2)),
    ],
)
def _sc_kernel(x_hbm, t_hbm, out_hbm, x_v0, x_v1, t_v0, t_v1, res_v, sems):
    wid = lax.axis_index("s") * _NC + lax.axis_index("c")
    x_bufs, t_bufs = (x_v0, x_v1), (t_v0, t_v1)

    def issue(j, slot):
        p = j * _NW + wid
        b = lax.bitwise_and(p, _B - 1)
        h = lax.shift_right_logical(p, 3) + _H_TC
        cx = pltpu.async_copy(x_hbm.at[b, :, h, :], x_bufs[slot], sems.at[slot, 0])
        ct = pltpu.async_copy(t_hbm.at[b, h, :], t_bufs[slot], sems.at[slot, 1])
        return cx, ct

    lanes = lax.broadcasted_iota(jnp.int32, (_L,), 0)

    def compute(slot, carry):
        x_v, t_v = x_bufs[slot], t_bufs[slot]

        def chunk(i, carry2):
            a1s, a2s, a3 = carry2
            a1s, a2s = list(a1s), list(a2s)
            col = lanes + i * _L
            tw = t_v[pl.ds(i * _L, _L)]
            xg = plsc.load_gather(x_v, [tw, col])
            a3 = a3 + xg
            for c in range(_C):
                xv = x_v[c, pl.ds(i * _L, _L)]
                nx = -xv
                na = jnp.minimum(xv, nx)
                u = jnp.exp(na)
                pp = _LP4[0]
                for k in _LP4[1:]:
                    pp = pp * u + k
                a1s[c & 3] = a1s[c & 3] + (xv - na)
                a2s[c & 3] = a2s[c & 3] + pp
            return tuple(a1s), tuple(a2s), a3

        return lax.fori_loop(0, _W // _L, chunk, carry)

    z = jnp.zeros((_L,), jnp.float32)
    carry = ((z, z, z, z), (z, z, z, z), z)
    pending = issue(0, 0)
    for j in range(_PPW):
        nxt = issue(j + 1, (j + 1) & 1) if j + 1 < _PPW else None
        pending[0].wait()
        pending[1].wait()
        carry = compute(j & 1, carry)
        pending = nxt
    a1s, a2s, a3 = carry
    acc1 = (a1s[0] + a1s[1]) + (a1s[2] + a1s[3])
    acc2 = (a2s[0] + a2s[1]) + (a2s[2] + a2s[3])
    res_v[...] = acc1 * 0.5 - a3 + acc2
    pltpu.sync_copy(res_v, out_hbm.at[wid])


def kernel(input, target, epoch):
    del epoch
    n = input.size
    tc_out = pl.pallas_call(
        _tc_body,
        grid=(_B, _H_TC // _HB),
        in_specs=[
            pl.BlockSpec((1, _C, _HB, _W), lambda b, h: (b, 0, h, 0)),
            pl.BlockSpec((1, _HB, _W), lambda b, h: (b, h, 0)),
        ],
        out_specs=pl.BlockSpec((8, _W), lambda b, h: (0, 0)),
        out_shape=jax.ShapeDtypeStruct((8, _W), jnp.float32),
    )(input, target)
    sc_out = _sc_kernel(input, target)
    return (jnp.sum(tc_out) + jnp.sum(sc_out)) / n


# speedup vs baseline: 1.7592x; 1.6975x over previous
import functools

import jax
import jax.numpy as jnp
from jax import lax
from jax.experimental import pallas as pl
from jax.experimental.pallas import tpu as pltpu
from jax.experimental.pallas import tpu_sc as plsc

_B, _C, _H, _W = 8, 19, 512, 512
_HB = 128
_H_TC = 512
assert _H_TC % _HB == 0

_NEG_LOG2E = -1.4426950408889634
_LN2 = 0.6931471805599453

_LP = (0.03044900453868939, -0.13158182508881333, 0.28527268109062165,
       -0.4902307234234269, 0.9992354838332771, 9.975032552123064e-06)

_LP4 = (-0.055459313742082655, 0.21866548366222538, -0.46644243862756857,
        0.9962619482337944, 6.94457445418184e-05)


def _tc_body(x_ref, t_ref, out_ref):
    def chunk(i, carry):
        acc1, acc2 = carry
        r = i * 8
        t = t_ref[0, pl.ds(r, 8), :]
        for c in range(_C):
            xc = x_ref[0, c, pl.ds(r, 8), :]
            a = jnp.abs(xc)
            zpa = jnp.where(t == c, a - xc, a + xc)
            u = jnp.exp2(a * _NEG_LOG2E)
            l = jnp.log2(1.0 + u)
            acc1 = acc1 + zpa
            acc2 = acc2 + l
        return acc1, acc2

    z = jnp.zeros((8, _W), jnp.float32)
    acc1, acc2 = jax.lax.fori_loop(0, _HB // 8, chunk, (z, z))

    @pl.when((pl.program_id(0) == 0) & (pl.program_id(1) == 0))
    def _():
        out_ref[...] = jnp.zeros_like(out_ref)

    out_ref[...] += 0.5 * acc1 + _LN2 * acc2


_info = plsc.get_sparse_core_info()
_NC, _NS, _L = _info.num_cores, _info.num_subcores, _info.num_lanes
_NW = _NC * _NS
_HSC = _H - _H_TC
_NPAIRS = _B * _HSC
_PPW = _NPAIRS // _NW

_sc_mesh = plsc.VectorSubcoreMesh(core_axis_name="c", subcore_axis_name="s")


@functools.partial(
    pl.kernel,
    mesh=_sc_mesh,
    compiler_params=pltpu.CompilerParams(needs_layout_passes=False),
    out_type=jax.ShapeDtypeStruct((_NW, _L), jnp.float32),
    scratch_types=[
        pltpu.VMEM((_C, _W), jnp.float32),
        pltpu.VMEM((_C, _W), jnp.float32),
        pltpu.VMEM((_W,), jnp.int32),
        pltpu.VMEM((_W,), jnp.int32),
        pltpu.VMEM((_L,), jnp.float32),
        pltpu.SemaphoreType.DMA((2, 2)),
    ],
)
def _sc_kernel(x_hbm, t_hbm, out_hbm, x_v0, x_v1, t_v0, t_v1, res_v, sems):
    wid = lax.axis_index("s") * _NC + lax.axis_index("c")
    x_bufs, t_bufs = (x_v0, x_v1), (t_v0, t_v1)

    def issue(j, slot):
        p = j * _NW + wid
        b = lax.bitwise_and(p, _B - 1)
        h = lax.shift_right_logical(p, 3) + _H_TC
        cx = pltpu.async_copy(x_hbm.at[b, :, h, :], x_bufs[slot], sems.at[slot, 0])
        ct = pltpu.async_copy(t_hbm.at[b, h, :], t_bufs[slot], sems.at[slot, 1])
        return cx, ct

    lanes = lax.broadcasted_iota(jnp.int32, (_L,), 0)

    def compute(slot, carry):
        x_v, t_v = x_bufs[slot], t_bufs[slot]

        def chunk(i, carry2):
            a1s, a2s, a3 = carry2
            a1s, a2s = list(a1s), list(a2s)
            col = lanes + i * _L
            tw = t_v[pl.ds(i * _L, _L)]
            xg = plsc.load_gather(x_v, [tw, col])
            a3 = a3 + xg
            for c in range(_C):
                xv = x_v[c, pl.ds(i * _L, _L)]
                nx = -xv
                na = jnp.minimum(xv, nx)
                u = jnp.exp(na)
                pp = _LP4[0]
                for k in _LP4[1:]:
                    pp = pp * u + k
                a1s[c & 3] = a1s[c & 3] + (xv - na)
                a2s[c & 3] = a2s[c & 3] + pp
            return tuple(a1s), tuple(a2s), a3

        return lax.fori_loop(0, _W // _L, chunk, carry)

    z = jnp.zeros((_L,), jnp.float32)
    carry = ((z, z, z, z), (z, z, z, z), z)
    pending = issue(0, 0)
    for j in range(_PPW):
        nxt = issue(j + 1, (j + 1) & 1) if j + 1 < _PPW else None
        pending[0].wait()
        pending[1].wait()
        carry = compute(j & 1, carry)
        pending = nxt
    a1s, a2s, a3 = carry
    acc1 = (a1s[0] + a1s[1]) + (a1s[2] + a1s[3])
    acc2 = (a2s[0] + a2s[1]) + (a2s[2] + a2s[3])
    res_v[...] = acc1 * 0.5 - a3 + acc2
    pltpu.sync_copy(res_v, out_hbm.at[wid])


def kernel(input, target, epoch):
    del epoch
    n = input.size
    tc_out = pl.pallas_call(
        _tc_body,
        grid=(_B, _H_TC // _HB),
        in_specs=[
            pl.BlockSpec((1, _C, _HB, _W), lambda b, h: (b, 0, h, 0)),
            pl.BlockSpec((1, _HB, _W), lambda b, h: (b, h, 0)),
        ],
        out_specs=pl.BlockSpec((8, _W), lambda b, h: (0, 0)),
        out_shape=jax.ShapeDtypeStruct((8, _W), jnp.float32),
    )(input, target)
    sc_out = _sc_kernel(input, target) if _HSC else jnp.zeros((), jnp.float32)
    return (jnp.sum(tc_out) + jnp.sum(sc_out)) / n
